# baseline (speedup 1.0000x reference)
import functools

import jax
import jax.numpy as jnp
from jax import lax
from jax.experimental import pallas as pl
from jax.experimental.pallas import tpu as pltpu
from jax.experimental.pallas import tpu_sc as plsc

E = 320000
N = 10000
H = 4
C = 32
D = 128
HC = H * C

ROW = 8
CHUNK = 128
NCHUNK = E // CHUNK
NW = 32
KFULL = NCHUNK // NW
NREM = NCHUNK - KFULL * NW
RPT = 632
RPT_LAST = N - 15 * RPT

BE = 2560
GRID_E = E // BE

_sc_mesh = plsc.VectorSubcoreMesh(core_axis_name="c", subcore_axis_name="s",
                                  num_cores=2, num_subcores=16)


def _k1_body(xi, xj, ea, ws, wt, we, attn, sel, ez_out):
    u = jnp.dot(xj[...], wt[...], preferred_element_type=jnp.float32)
    u = u + jnp.dot(xi[...], ws[...], preferred_element_type=jnp.float32)
    u = u + jnp.dot(ea[...], we[...], preferred_element_type=jnp.float32)
    u = jnp.where(u >= 0.0, u, 0.01 * u) * attn[...]
    z = jnp.dot(u, sel[...], preferred_element_type=jnp.float32)
    col = lax.broadcasted_iota(jnp.int32, z.shape, 1)
    ez_out[...] = jnp.where(col < H, jnp.exp(z), 0.0)


@functools.partial(
    pl.kernel,
    out_type=jax.ShapeDtypeStruct((2, N, ROW), jnp.float32),
    mesh=_sc_mesh,
    compiler_params=pltpu.CompilerParams(use_tc_tiling_on_sc=False),
    scratch_types=[
        pltpu.VMEM((CHUNK,), jnp.int32),
        pltpu.VMEM((CHUNK, ROW), jnp.float32),
        pltpu.VMEM_SHARED((N, ROW), jnp.float32),
    ],
)
def _k2(send_hbm, ezp_hbm, zer8_hbm, part_hbm, idx_v, pay_v, acc_sh):
    c = lax.axis_index("c")
    s = lax.axis_index("s")
    wid = s * 2 + c

    @pl.when(s < 15)
    def _():
        pltpu.sync_copy(zer8_hbm.at[pl.ds(s * RPT, RPT)],
                        acc_sh.at[pl.ds(s * RPT, RPT)])

    @pl.when(s == 15)
    def _():
        pltpu.sync_copy(zer8_hbm.at[pl.ds(15 * RPT, RPT_LAST)],
                        acc_sh.at[pl.ds(15 * RPT, RPT_LAST)])

    plsc.subcore_barrier()

    def _one(chunk):
        off = chunk * CHUNK
        pltpu.sync_copy(send_hbm.at[pl.ds(off, CHUNK)], idx_v)
        pltpu.sync_copy(ezp_hbm.at[pl.ds(off, CHUNK), :], pay_v)
        pltpu.sync_copy(pay_v, acc_sh.at[idx_v], add=True)

    def _body(k, _):
        _one(k * NW + wid)
        return _

    lax.fori_loop(0, KFULL, _body, 0)

    @pl.when(wid < NREM)
    def _():
        _one(KFULL * NW + wid)

    plsc.subcore_barrier()

    @pl.when(s < 15)
    def _():
        pltpu.sync_copy(acc_sh.at[pl.ds(s * RPT, RPT)],
                        part_hbm.at[c, pl.ds(s * RPT, RPT), :])

    @pl.when(s == 15)
    def _():
        pltpu.sync_copy(acc_sh.at[pl.ds(15 * RPT, RPT_LAST)],
                        part_hbm.at[c, pl.ds(15 * RPT, RPT_LAST), :])


def _k2b_body(p, o):
    o[...] = p[0] + p[1]


@functools.partial(
    pl.kernel,
    out_type=jax.ShapeDtypeStruct((E, ROW), jnp.float32),
    mesh=_sc_mesh,
    compiler_params=pltpu.CompilerParams(use_tc_tiling_on_sc=False),
    scratch_types=[
        pltpu.VMEM((CHUNK,), jnp.int32),
        pltpu.VMEM((CHUNK, ROW), jnp.float32),
        pltpu.SemaphoreType.DMA,
    ],
)
def _k3(send_hbm, denom_hbm, dg_hbm, idx_v, rows_v, sem):
    c = lax.axis_index("c")
    s = lax.axis_index("s")
    wid = s * 2 + c

    def _one(chunk):
        off = chunk * CHUNK
        pltpu.sync_copy(send_hbm.at[pl.ds(off, CHUNK)], idx_v)
        pltpu.async_copy(denom_hbm.at[idx_v], rows_v, sem).wait()
        pltpu.sync_copy(rows_v, dg_hbm.at[pl.ds(off, CHUNK), :])

    def _body(k, _):
        _one(k * NW + wid)
        return _

    lax.fori_loop(0, KFULL, _body, 0)

    @pl.when(wid < NREM)
    def _():
        _one(KFULL * NW + wid)


def _k4a_body(xj, wt, t_out):
    t_out[...] = jnp.dot(xj[...], wt[...], preferred_element_type=jnp.float32)


def _k4_body(t_ref, ez, dg, hexp, hsum, m_out):
    t = t_ref[...]
    col = lax.broadcasted_iota(jnp.int32, ez.shape, 1)
    a = jnp.where(col < H, ez[...] / dg[...], 0.0)
    aexp = jnp.dot(a, hexp[...], preferred_element_type=jnp.float32)
    m_out[...] = jnp.dot(t * aexp, hsum[...], preferred_element_type=jnp.float32)


NCH_SC = 16
KFULL5 = NCHUNK // 16
NREM5 = NCHUNK - KFULL5 * 16


@functools.partial(
    pl.kernel,
    out_type=jax.ShapeDtypeStruct((N, C), jnp.float32),
    mesh=_sc_mesh,
    compiler_params=pltpu.CompilerParams(use_tc_tiling_on_sc=False),
    scratch_types=[
        pltpu.VMEM((CHUNK,), jnp.int32),
        pltpu.VMEM((CHUNK, NCH_SC), jnp.float32),
        pltpu.VMEM_SHARED((N, NCH_SC), jnp.float32),
    ],
)
def _k5(recv_hbm, m_hbm, zer16_hbm, aggr_hbm, idx_v, pay_v, acc_sh):
    c = lax.axis_index("c")
    s = lax.axis_index("s")
    colbase = c * NCH_SC

    @pl.when(s < 15)
    def _():
        pltpu.sync_copy(zer16_hbm.at[pl.ds(s * RPT, RPT)],
                        acc_sh.at[pl.ds(s * RPT, RPT)])

    @pl.when(s == 15)
    def _():
        pltpu.sync_copy(zer16_hbm.at[pl.ds(15 * RPT, RPT_LAST)],
                        acc_sh.at[pl.ds(15 * RPT, RPT_LAST)])

    plsc.subcore_barrier()

    def _one(chunk):
        off = chunk * CHUNK
        pltpu.sync_copy(recv_hbm.at[pl.ds(off, CHUNK)], idx_v)
        pltpu.sync_copy(m_hbm.at[pl.ds(off, CHUNK), pl.ds(colbase, NCH_SC)], pay_v)
        pltpu.sync_copy(pay_v, acc_sh.at[idx_v], add=True)

    def _body(k, _):
        _one(k * 16 + s)
        return _

    lax.fori_loop(0, KFULL5, _body, 0)

    @pl.when(s < NREM5)
    def _():
        _one(KFULL5 * 16 + s)

    plsc.subcore_barrier()

    @pl.when(s < 15)
    def _():
        pltpu.sync_copy(acc_sh.at[pl.ds(s * RPT, RPT)],
                        aggr_hbm.at[pl.ds(s * RPT, RPT), pl.ds(colbase, NCH_SC)])

    @pl.when(s == 15)
    def _():
        pltpu.sync_copy(acc_sh.at[pl.ds(15 * RPT, RPT_LAST)],
                        aggr_hbm.at[pl.ds(15 * RPT, RPT_LAST), pl.ds(colbase, NCH_SC)])


def kernel(x_i, x_j, edge_attribute, senders, receivers, Ws, Wt, We, attn):
    f32 = jnp.float32
    attn_flat = attn.reshape(1, HC)
    colid = jnp.arange(HC, dtype=jnp.int32)
    sel = (colid[:, None] // C == jnp.arange(ROW, dtype=jnp.int32)[None, :]).astype(f32)
    hexp = sel.T
    hsum = ((colid % C)[:, None] ==
            jnp.arange(C, dtype=jnp.int32)[None, :]).astype(f32) * (1.0 / H)

    wspec = pl.BlockSpec((D, HC), lambda i: (0, 0))
    ezp = pl.pallas_call(
        _k1_body,
        grid=(GRID_E,),
        in_specs=[
            pl.BlockSpec((BE, D), lambda i: (i, 0)),
            pl.BlockSpec((BE, D), lambda i: (i, 0)),
            pl.BlockSpec((BE, D), lambda i: (i, 0)),
            wspec, wspec, wspec,
            pl.BlockSpec((1, HC), lambda i: (0, 0)),
            pl.BlockSpec((HC, ROW), lambda i: (0, 0)),
        ],
        out_specs=pl.BlockSpec((BE, ROW), lambda i: (i, 0)),
        out_shape=jax.ShapeDtypeStruct((E, ROW), f32),
    )(x_i, x_j, edge_attribute, Ws, Wt, We, attn_flat, sel)

    t_full = pl.pallas_call(
        _k4a_body,
        grid=(GRID_E,),
        in_specs=[pl.BlockSpec((BE, D), lambda i: (i, 0)), wspec],
        out_specs=pl.BlockSpec((BE, D), lambda i: (i, 0)),
        out_shape=jax.ShapeDtypeStruct((E, D), f32),
    )(x_j, Wt)

    zer8 = jnp.zeros((N, ROW), f32)
    zer16 = jnp.zeros((N, NCH_SC), f32)

    partials = _k2(senders, ezp, zer8)

    denom = pl.pallas_call(
        _k2b_body,
        out_shape=jax.ShapeDtypeStruct((N * ROW // D, D), f32),
    )(partials.reshape(2, N * ROW // D, D)).reshape(N, ROW)

    dg = _k3(senders, denom)

    m = pl.pallas_call(
        _k4_body,
        grid=(GRID_E,),
        in_specs=[
            pl.BlockSpec((BE, D), lambda i: (i, 0)),
            pl.BlockSpec((BE, ROW), lambda i: (i, 0)),
            pl.BlockSpec((BE, ROW), lambda i: (i, 0)),
            pl.BlockSpec((ROW, HC), lambda i: (0, 0)),
            pl.BlockSpec((HC, C), lambda i: (0, 0)),
        ],
        out_specs=pl.BlockSpec((BE, C), lambda i: (i, 0)),
        out_shape=jax.ShapeDtypeStruct((E, C), f32),
    )(t_full, ezp, dg, hexp, hsum)

    aggr = _k5(receivers, m, zer16)

    return (aggr, m)


# speedup vs baseline: 1.0628x; 1.0628x over previous
import functools

import jax
import jax.numpy as jnp
from jax import lax
from jax.experimental import pallas as pl
from jax.experimental.pallas import tpu as pltpu
from jax.experimental.pallas import tpu_sc as plsc

E = 320000
N = 10000
H = 4
C = 32
D = 128
HC = H * C

ROW = 8
CHUNK = 128
NCHUNK = E // CHUNK
NW = 32
KFULL = NCHUNK // NW
NREM = NCHUNK - KFULL * NW
RPT = 632
RPT_LAST = N - 15 * RPT

BE = 2560
GRID_E = E // BE

_sc_mesh = plsc.VectorSubcoreMesh(core_axis_name="c", subcore_axis_name="s",
                                  num_cores=2, num_subcores=16)


def _k1_body(xi, xj, ea, ws, wt, we, attn, sel, ez_out):
    u = jnp.dot(xj[...], wt[...], preferred_element_type=jnp.float32)
    u = u + jnp.dot(xi[...], ws[...], preferred_element_type=jnp.float32)
    u = u + jnp.dot(ea[...], we[...], preferred_element_type=jnp.float32)
    u = jnp.where(u >= 0.0, u, 0.01 * u) * attn[...]
    z = jnp.dot(u, sel[...], preferred_element_type=jnp.float32)
    col = lax.broadcasted_iota(jnp.int32, z.shape, 1)
    ez_out[...] = jnp.where(col < H, jnp.exp(z), 0.0)


@functools.partial(
    pl.kernel,
    out_type=jax.ShapeDtypeStruct((2, N, ROW), jnp.float32),
    mesh=_sc_mesh,
    compiler_params=pltpu.CompilerParams(use_tc_tiling_on_sc=False),
    scratch_types=[
        pltpu.VMEM((CHUNK,), jnp.int32),
        pltpu.VMEM((CHUNK, ROW), jnp.float32),
        pltpu.VMEM_SHARED((N, ROW), jnp.float32),
    ],
)
def _k2(send_hbm, ezp_hbm, zer8_hbm, part_hbm, idx_v, pay_v, acc_sh):
    c = lax.axis_index("c")
    s = lax.axis_index("s")
    wid = s * 2 + c

    @pl.when(s < 15)
    def _():
        pltpu.sync_copy(zer8_hbm.at[pl.ds(s * RPT, RPT)],
                        acc_sh.at[pl.ds(s * RPT, RPT)])

    @pl.when(s == 15)
    def _():
        pltpu.sync_copy(zer8_hbm.at[pl.ds(15 * RPT, RPT_LAST)],
                        acc_sh.at[pl.ds(15 * RPT, RPT_LAST)])

    plsc.subcore_barrier()

    def _one(chunk):
        off = chunk * CHUNK
        pltpu.sync_copy(send_hbm.at[pl.ds(off, CHUNK)], idx_v)
        pltpu.sync_copy(ezp_hbm.at[pl.ds(off, CHUNK), :], pay_v)
        pltpu.sync_copy(pay_v, acc_sh.at[idx_v], add=True)

    def _body(k, _):
        _one(k * NW + wid)
        return _

    lax.fori_loop(0, KFULL, _body, 0)

    @pl.when(wid < NREM)
    def _():
        _one(KFULL * NW + wid)

    plsc.subcore_barrier()

    @pl.when(s < 15)
    def _():
        pltpu.sync_copy(acc_sh.at[pl.ds(s * RPT, RPT)],
                        part_hbm.at[c, pl.ds(s * RPT, RPT), :])

    @pl.when(s == 15)
    def _():
        pltpu.sync_copy(acc_sh.at[pl.ds(15 * RPT, RPT_LAST)],
                        part_hbm.at[c, pl.ds(15 * RPT, RPT_LAST), :])


def _k2b_body(p, o):
    o[...] = p[0] + p[1]


EPT = E // NW
CH3 = 2000
NCH3 = EPT // CH3
NG3 = CH3 // 16


@functools.partial(
    pl.kernel,
    out_type=jax.ShapeDtypeStruct((E * ROW,), jnp.float32),
    mesh=_sc_mesh,
    compiler_params=pltpu.CompilerParams(use_tc_tiling_on_sc=False,
                                         needs_layout_passes=False),
    scratch_types=[
        pltpu.VMEM((N * ROW,), jnp.float32),
        pltpu.VMEM((CH3,), jnp.int32),
        pltpu.VMEM((CH3 * ROW,), jnp.float32),
    ],
)
def _k3(send_hbm, denf_hbm, dg_hbm, den_v, sidx_v, out_v):
    c = lax.axis_index("c")
    s = lax.axis_index("s")
    wid = s * 2 + c
    pltpu.sync_copy(denf_hbm, den_v)
    lane8 = lax.iota(jnp.int32, 16) * ROW
    for ch in range(NCH3):
        ebase = wid * EPT + ch * CH3
        pltpu.sync_copy(send_hbm.at[pl.ds(ebase, CH3)], sidx_v)

        def _grp(g, _):
            sv = sidx_v[pl.ds(g * 16, 16)] * ROW
            ov = g * (16 * ROW) + lane8
            for j in range(ROW):
                gj = plsc.load_gather(den_v, (sv + j,))
                plsc.store_scatter(out_v, (ov + j,), gj)
            return _

        lax.fori_loop(0, NG3, _grp, 0)
        pltpu.sync_copy(out_v, dg_hbm.at[pl.ds(ebase * ROW, CH3 * ROW)])


def _k4_body(xj, wt, ez, dg, hexp, hsum, m_out):
    t = jnp.dot(xj[...], wt[...], preferred_element_type=jnp.float32)
    col = lax.broadcasted_iota(jnp.int32, ez.shape, 1)
    a = jnp.where(col < H, ez[...] / dg[...], 0.0)
    aexp = jnp.dot(a, hexp[...], preferred_element_type=jnp.float32)
    m_out[...] = jnp.dot(t * aexp, hsum[...], preferred_element_type=jnp.float32)


NCH_SC = 16
KFULL5 = NCHUNK // 16
NREM5 = NCHUNK - KFULL5 * 16


@functools.partial(
    pl.kernel,
    out_type=jax.ShapeDtypeStruct((N, C), jnp.float32),
    mesh=_sc_mesh,
    compiler_params=pltpu.CompilerParams(use_tc_tiling_on_sc=False),
    scratch_types=[
        pltpu.VMEM((CHUNK,), jnp.int32),
        pltpu.VMEM((CHUNK, NCH_SC), jnp.float32),
        pltpu.VMEM_SHARED((N, NCH_SC), jnp.float32),
    ],
)
def _k5(recv_hbm, m_hbm, zer16_hbm, aggr_hbm, idx_v, pay_v, acc_sh):
    c = lax.axis_index("c")
    s = lax.axis_index("s")
    colbase = c * NCH_SC

    @pl.when(s < 15)
    def _():
        pltpu.sync_copy(zer16_hbm.at[pl.ds(s * RPT, RPT)],
                        acc_sh.at[pl.ds(s * RPT, RPT)])

    @pl.when(s == 15)
    def _():
        pltpu.sync_copy(zer16_hbm.at[pl.ds(15 * RPT, RPT_LAST)],
                        acc_sh.at[pl.ds(15 * RPT, RPT_LAST)])

    plsc.subcore_barrier()

    def _one(chunk):
        off = chunk * CHUNK
        pltpu.sync_copy(recv_hbm.at[pl.ds(off, CHUNK)], idx_v)
        pltpu.sync_copy(m_hbm.at[pl.ds(off, CHUNK), pl.ds(colbase, NCH_SC)], pay_v)
        pltpu.sync_copy(pay_v, acc_sh.at[idx_v], add=True)

    def _body(k, _):
        _one(k * 16 + s)
        return _

    lax.fori_loop(0, KFULL5, _body, 0)

    @pl.when(s < NREM5)
    def _():
        _one(KFULL5 * 16 + s)

    plsc.subcore_barrier()

    @pl.when(s < 15)
    def _():
        pltpu.sync_copy(acc_sh.at[pl.ds(s * RPT, RPT)],
                        aggr_hbm.at[pl.ds(s * RPT, RPT), pl.ds(colbase, NCH_SC)])

    @pl.when(s == 15)
    def _():
        pltpu.sync_copy(acc_sh.at[pl.ds(15 * RPT, RPT_LAST)],
                        aggr_hbm.at[pl.ds(15 * RPT, RPT_LAST), pl.ds(colbase, NCH_SC)])


def kernel(x_i, x_j, edge_attribute, senders, receivers, Ws, Wt, We, attn):
    f32 = jnp.float32
    attn_flat = attn.reshape(1, HC)
    colid = jnp.arange(HC, dtype=jnp.int32)
    sel = (colid[:, None] // C == jnp.arange(ROW, dtype=jnp.int32)[None, :]).astype(f32)
    hexp = sel.T
    hsum = ((colid % C)[:, None] ==
            jnp.arange(C, dtype=jnp.int32)[None, :]).astype(f32) * (1.0 / H)

    wspec = pl.BlockSpec((D, HC), lambda i: (0, 0))
    ezp = pl.pallas_call(
        _k1_body,
        grid=(GRID_E,),
        in_specs=[
            pl.BlockSpec((BE, D), lambda i: (i, 0)),
            pl.BlockSpec((BE, D), lambda i: (i, 0)),
            pl.BlockSpec((BE, D), lambda i: (i, 0)),
            wspec, wspec, wspec,
            pl.BlockSpec((1, HC), lambda i: (0, 0)),
            pl.BlockSpec((HC, ROW), lambda i: (0, 0)),
        ],
        out_specs=pl.BlockSpec((BE, ROW), lambda i: (i, 0)),
        out_shape=jax.ShapeDtypeStruct((E, ROW), f32),
    )(x_i, x_j, edge_attribute, Ws, Wt, We, attn_flat, sel)

    zer8 = jnp.zeros((N, ROW), f32)
    zer16 = jnp.zeros((N, NCH_SC), f32)

    partials = _k2(senders, ezp, zer8)

    denf = pl.pallas_call(
        _k2b_body,
        out_shape=jax.ShapeDtypeStruct((N * ROW // D, D), f32),
    )(partials.reshape(2, N * ROW // D, D)).reshape(N * ROW)

    dg = _k3(senders, denf).reshape(E, ROW)

    m = pl.pallas_call(
        _k4_body,
        grid=(GRID_E,),
        in_specs=[
            pl.BlockSpec((BE, D), lambda i: (i, 0)),
            wspec,
            pl.BlockSpec((BE, ROW), lambda i: (i, 0)),
            pl.BlockSpec((BE, ROW), lambda i: (i, 0)),
            pl.BlockSpec((ROW, HC), lambda i: (0, 0)),
            pl.BlockSpec((HC, C), lambda i: (0, 0)),
        ],
        out_specs=pl.BlockSpec((BE, C), lambda i: (i, 0)),
        out_shape=jax.ShapeDtypeStruct((E, C), f32),
    )(x_j, Wt, ezp, dg, hexp, hsum)

    aggr = _k5(receivers, m, zer16)

    return (aggr, m)


# speedup vs baseline: 1.3710x; 1.2900x over previous
import functools

import jax
import jax.numpy as jnp
from jax import lax
from jax.experimental import pallas as pl
from jax.experimental.pallas import tpu as pltpu
from jax.experimental.pallas import tpu_sc as plsc

E = 320000
N = 10000
H = 4
C = 32
D = 128
HC = H * C

ROW = 8
CHUNK = 128
NG = CHUNK // 16
NCHUNK = E // CHUNK
NW = 32
KFULL = NCHUNK // NW
NREM = NCHUNK - KFULL * NW
RPT = 632
RPT_LAST = N - 15 * RPT

BE = 2560
GRID_E = E // BE

_sc_mesh = plsc.VectorSubcoreMesh(core_axis_name="c", subcore_axis_name="s",
                                  num_cores=2, num_subcores=16)
_sc_params = pltpu.CompilerParams(use_tc_tiling_on_sc=False,
                                  needs_layout_passes=False)


def _k1_body(xi, xj, ea, ws, wt, we, attn, selt, ez_out):
    u = jnp.dot(xj[...], wt[...], preferred_element_type=jnp.float32)
    u = u + jnp.dot(xi[...], ws[...], preferred_element_type=jnp.float32)
    u = u + jnp.dot(ea[...], we[...], preferred_element_type=jnp.float32)
    u = jnp.where(u >= 0.0, u, 0.01 * u) * attn[...]
    zt = lax.dot_general(selt[...], u, (((1,), (1,)), ((), ())),
                         preferred_element_type=jnp.float32)
    row = lax.broadcasted_iota(jnp.int32, zt.shape, 0)
    ez_out[...] = jnp.where(row < H, jnp.exp(zt), 0.0)


@functools.partial(
    pl.kernel,
    out_type=jax.ShapeDtypeStruct((2, N, ROW), jnp.float32),
    mesh=_sc_mesh,
    compiler_params=_sc_params,
    scratch_types=[
        pltpu.VMEM((CHUNK,), jnp.int32),
        pltpu.VMEM((ROW, CHUNK), jnp.float32),
        pltpu.VMEM((CHUNK, ROW), jnp.float32),
        pltpu.VMEM_SHARED((N, ROW), jnp.float32),
    ],
)
def _k2(send_hbm, ezh_hbm, zer8_hbm, part_hbm, idx_v, buf_v, pay_v, acc_sh):
    c = lax.axis_index("c")
    s = lax.axis_index("s")
    wid = s * 2 + c
    lane = lax.iota(jnp.int32, 16)

    @pl.when(s < 15)
    def _():
        pltpu.sync_copy(zer8_hbm.at[pl.ds(s * RPT, RPT)],
                        acc_sh.at[pl.ds(s * RPT, RPT)])

    @pl.when(s == 15)
    def _():
        pltpu.sync_copy(zer8_hbm.at[pl.ds(15 * RPT, RPT_LAST)],
                        acc_sh.at[pl.ds(15 * RPT, RPT_LAST)])

    plsc.subcore_barrier()

    def _one(chunk):
        off = chunk * CHUNK
        pltpu.sync_copy(send_hbm.at[pl.ds(off, CHUNK)], idx_v)
        pltpu.sync_copy(ezh_hbm.at[:, pl.ds(off, CHUNK)], buf_v)

        def _rp(g, carry):
            rvec = g * 16 + lane
            for j in range(ROW):
                v = buf_v[j, pl.ds(g * 16, 16)]
                plsc.store_scatter(pay_v,
                                   (rvec, jnp.full((16,), j, jnp.int32)), v)
            return carry

        lax.fori_loop(0, NG, _rp, 0)
        pltpu.sync_copy(pay_v, acc_sh.at[idx_v], add=True)

    def _body(k, carry):
        _one(k * NW + wid)
        return carry

    lax.fori_loop(0, KFULL, _body, 0)

    @pl.when(wid < NREM)
    def _():
        _one(KFULL * NW + wid)

    plsc.subcore_barrier()

    @pl.when(s < 15)
    def _():
        pltpu.sync_copy(acc_sh.at[pl.ds(s * RPT, RPT)],
                        part_hbm.at[c, pl.ds(s * RPT, RPT), :])

    @pl.when(s == 15)
    def _():
        pltpu.sync_copy(acc_sh.at[pl.ds(15 * RPT, RPT_LAST)],
                        part_hbm.at[c, pl.ds(15 * RPT, RPT_LAST), :])


def _k2b_body(p, o):
    o[...] = p[0] + p[1]


EPT = E // NW
CH3 = 2000
NCH3 = EPT // CH3
NG3 = CH3 // 16


@functools.partial(
    pl.kernel,
    out_type=jax.ShapeDtypeStruct((ROW, E), jnp.float32),
    mesh=_sc_mesh,
    compiler_params=_sc_params,
    scratch_types=[
        pltpu.VMEM((N * ROW,), jnp.float32),
        pltpu.VMEM((CH3,), jnp.int32),
        pltpu.VMEM((ROW, CH3), jnp.float32),
    ],
)
def _k3(send_hbm, denf_hbm, dgh_hbm, den_v, sidx_v, out_v):
    c = lax.axis_index("c")
    s = lax.axis_index("s")
    wid = s * 2 + c
    pltpu.sync_copy(denf_hbm, den_v)
    for ch in range(NCH3):
        ebase = wid * EPT + ch * CH3
        pltpu.sync_copy(send_hbm.at[pl.ds(ebase, CH3)], sidx_v)

        def _grp(g, carry):
            sv = sidx_v[pl.ds(g * 16, 16)] * ROW
            for j in range(ROW):
                gj = plsc.load_gather(den_v, (sv + j,))
                out_v[j, pl.ds(g * 16, 16)] = gj
            return carry

        lax.fori_loop(0, NG3, _grp, 0)
        pltpu.sync_copy(out_v, dgh_hbm.at[:, pl.ds(ebase, CH3)])


def _k4_body(xj, wt, ezh, dgh, hexp, hsum, m_out, mcm_out):
    t = jnp.dot(xj[...], wt[...], preferred_element_type=jnp.float32)
    row = lax.broadcasted_iota(jnp.int32, ezh.shape, 0)
    a = jnp.where(row < H, ezh[...] / dgh[...], 0.0)
    aexp = lax.dot_general(a, hexp[...], (((0,), (0,)), ((), ())),
                           preferred_element_type=jnp.float32)
    v = t * aexp
    m_out[...] = jnp.dot(v, hsum[...], preferred_element_type=jnp.float32)
    mcm_out[...] = lax.dot_general(hsum[...], v, (((0,), (1,)), ((), ())),
                                   preferred_element_type=jnp.float32)


NCH_SC = 16
KFULL5 = NCHUNK // 16
NREM5 = NCHUNK - KFULL5 * 16


@functools.partial(
    pl.kernel,
    out_type=jax.ShapeDtypeStruct((N, C), jnp.float32),
    mesh=_sc_mesh,
    compiler_params=_sc_params,
    scratch_types=[
        pltpu.VMEM((CHUNK,), jnp.int32),
        pltpu.VMEM((NCH_SC, CHUNK), jnp.float32),
        pltpu.VMEM((CHUNK, NCH_SC), jnp.float32),
        pltpu.VMEM_SHARED((N, NCH_SC), jnp.float32),
    ],
)
def _k5(recv_hbm, mcm_hbm, zer16_hbm, aggr_hbm, idx_v, buf_v, pay_v, acc_sh):
    c = lax.axis_index("c")
    s = lax.axis_index("s")
    colbase = c * NCH_SC
    lane = lax.iota(jnp.int32, 16)

    @pl.when(s < 15)
    def _():
        pltpu.sync_copy(zer16_hbm.at[pl.ds(s * RPT, RPT)],
                        acc_sh.at[pl.ds(s * RPT, RPT)])

    @pl.when(s == 15)
    def _():
        pltpu.sync_copy(zer16_hbm.at[pl.ds(15 * RPT, RPT_LAST)],
                        acc_sh.at[pl.ds(15 * RPT, RPT_LAST)])

    plsc.subcore_barrier()

    def _one(chunk):
        off = chunk * CHUNK
        pltpu.sync_copy(recv_hbm.at[pl.ds(off, CHUNK)], idx_v)
        pltpu.sync_copy(mcm_hbm.at[pl.ds(colbase, NCH_SC), pl.ds(off, CHUNK)],
                        buf_v)

        def _rp(g, carry):
            rvec = g * 16 + lane
            for j in range(NCH_SC):
                v = buf_v[j, pl.ds(g * 16, 16)]
                plsc.store_scatter(pay_v,
                                   (rvec, jnp.full((16,), j, jnp.int32)), v)
            return carry

        lax.fori_loop(0, NG, _rp, 0)
        pltpu.sync_copy(pay_v, acc_sh.at[idx_v], add=True)

    def _body(k, carry):
        _one(k * 16 + s)
        return carry

    lax.fori_loop(0, KFULL5, _body, 0)

    @pl.when(s < NREM5)
    def _():
        _one(KFULL5 * 16 + s)

    plsc.subcore_barrier()

    @pl.when(s < 15)
    def _():
        pltpu.sync_copy(acc_sh.at[pl.ds(s * RPT, RPT)],
                        aggr_hbm.at[pl.ds(s * RPT, RPT), pl.ds(colbase, NCH_SC)])

    @pl.when(s == 15)
    def _():
        pltpu.sync_copy(acc_sh.at[pl.ds(15 * RPT, RPT_LAST)],
                        aggr_hbm.at[pl.ds(15 * RPT, RPT_LAST),
                                    pl.ds(colbase, NCH_SC)])


def kernel(x_i, x_j, edge_attribute, senders, receivers, Ws, Wt, We, attn):
    f32 = jnp.float32
    attn_flat = attn.reshape(1, HC)
    colid = jnp.arange(HC, dtype=jnp.int32)
    selt = (colid[None, :] // C ==
            jnp.arange(ROW, dtype=jnp.int32)[:, None]).astype(f32)
    hexp = selt
    hsum = ((colid % C)[:, None] ==
            jnp.arange(C, dtype=jnp.int32)[None, :]).astype(f32) * (1.0 / H)

    wspec = pl.BlockSpec((D, HC), lambda i: (0, 0))
    espec = pl.BlockSpec((ROW, BE), lambda i: (0, i))
    ezh = pl.pallas_call(
        _k1_body,
        grid=(GRID_E,),
        in_specs=[
            pl.BlockSpec((BE, D), lambda i: (i, 0)),
            pl.BlockSpec((BE, D), lambda i: (i, 0)),
            pl.BlockSpec((BE, D), lambda i: (i, 0)),
            wspec, wspec, wspec,
            pl.BlockSpec((1, HC), lambda i: (0, 0)),
            pl.BlockSpec((ROW, HC), lambda i: (0, 0)),
        ],
        out_specs=espec,
        out_shape=jax.ShapeDtypeStruct((ROW, E), f32),
    )(x_i, x_j, edge_attribute, Ws, Wt, We, attn_flat, selt)

    zer8 = jnp.zeros((N, ROW), f32)
    zer16 = jnp.zeros((N, NCH_SC), f32)

    partials = _k2(senders, ezh, zer8)

    denf = pl.pallas_call(
        _k2b_body,
        out_shape=jax.ShapeDtypeStruct((N * ROW // D, D), f32),
    )(partials.reshape(2, N * ROW // D, D)).reshape(N * ROW)

    dgh = _k3(senders, denf)

    m, mcm = pl.pallas_call(
        _k4_body,
        grid=(GRID_E,),
        in_specs=[
            pl.BlockSpec((BE, D), lambda i: (i, 0)),
            wspec,
            espec,
            espec,
            pl.BlockSpec((ROW, HC), lambda i: (0, 0)),
            pl.BlockSpec((HC, C), lambda i: (0, 0)),
        ],
        out_specs=[
            pl.BlockSpec((BE, C), lambda i: (i, 0)),
            pl.BlockSpec((C, BE), lambda i: (0, i)),
        ],
        out_shape=[
            jax.ShapeDtypeStruct((E, C), f32),
            jax.ShapeDtypeStruct((C, E), f32),
        ],
    )(x_j, Wt, ezh, dgh, hexp, hsum)

    aggr = _k5(receivers, mcm, zer16)

    return (aggr, m)


# speedup vs baseline: 1.6029x; 1.1692x over previous
import functools

import jax
import jax.numpy as jnp
from jax import lax
from jax.experimental import pallas as pl
from jax.experimental.pallas import tpu as pltpu
from jax.experimental.pallas import tpu_sc as plsc

E = 320000
N = 10000
H = 4
C = 32
D = 128
HC = H * C

ROW = 8
CHUNK = 128
NG = CHUNK // 16
NCHUNK = E // CHUNK
NW = 32
KFULL = NCHUNK // NW
NREM = NCHUNK - KFULL * NW
RPT = 632
RPT_LAST = N - 15 * RPT

BE = 2560
GRID_E = E // BE

_sc_mesh = plsc.VectorSubcoreMesh(core_axis_name="c", subcore_axis_name="s",
                                  num_cores=2, num_subcores=16)
_sc_params = pltpu.CompilerParams(use_tc_tiling_on_sc=False,
                                  needs_layout_passes=False)


def _k1_body(xi, xj, ea, ws, wt, we, attn, selt, ez_out):
    u = jnp.dot(xj[...], wt[...], preferred_element_type=jnp.float32)
    u = u + jnp.dot(xi[...], ws[...], preferred_element_type=jnp.float32)
    u = u + jnp.dot(ea[...], we[...], preferred_element_type=jnp.float32)
    u = jnp.where(u >= 0.0, u, 0.01 * u) * attn[...]
    zt = lax.dot_general(selt[...], u, (((1,), (1,)), ((), ())),
                         preferred_element_type=jnp.float32)
    row = lax.broadcasted_iota(jnp.int32, zt.shape, 0)
    ez_out[...] = jnp.where(row < H, jnp.exp(zt), 0.0)


@functools.partial(
    pl.kernel,
    out_type=jax.ShapeDtypeStruct((2, N, ROW), jnp.float32),
    mesh=_sc_mesh,
    compiler_params=_sc_params,
    scratch_types=[
        pltpu.VMEM((2, CHUNK), jnp.int32),
        pltpu.VMEM((2, ROW, CHUNK), jnp.float32),
        pltpu.VMEM((2, CHUNK, ROW), jnp.float32),
        pltpu.VMEM_SHARED((N, ROW), jnp.float32),
        pltpu.SemaphoreType.DMA,
        pltpu.SemaphoreType.DMA,
        pltpu.SemaphoreType.DMA,
        pltpu.SemaphoreType.DMA,
    ],
)
def _k2(send_hbm, ezh_hbm, zer8_hbm, part_hbm, idx2, buf2, pay2, acc_sh,
        lsem0, lsem1, ssem0, ssem1):
    c = lax.axis_index("c")
    s = lax.axis_index("s")
    wid = s * 2 + c
    lane = lax.iota(jnp.int32, 16)
    lsem = (lsem0, lsem1)
    ssem = (ssem0, ssem1)

    @pl.when(s < 15)
    def _():
        pltpu.sync_copy(zer8_hbm.at[pl.ds(s * RPT, RPT)],
                        acc_sh.at[pl.ds(s * RPT, RPT)])

    @pl.when(s == 15)
    def _():
        pltpu.sync_copy(zer8_hbm.at[pl.ds(15 * RPT, RPT_LAST)],
                        acc_sh.at[pl.ds(15 * RPT, RPT_LAST)])

    plsc.subcore_barrier()

    def _repack(b):
        def _rp(g, carry):
            rvec = g * 16 + lane
            for j in range(ROW):
                v = buf2[b, j, pl.ds(g * 16, 16)]
                plsc.store_scatter(pay2.at[b],
                                   (rvec, jnp.full((16,), j, jnp.int32)), v)
            return carry

        lax.fori_loop(0, NG, _rp, 0)

    def _pair(g, carry):
        for b in range(2):
            k = 2 * g + b
            off = (k * NW + wid) * CHUNK

            @pl.when(g > 0)
            def _():
                pltpu.make_async_copy(pay2.at[b], acc_sh.at[idx2.at[b]],
                                      ssem[b]).wait()

            pltpu.async_copy(send_hbm.at[pl.ds(off, CHUNK)], idx2.at[b],
                             lsem[b])
            pltpu.async_copy(ezh_hbm.at[:, pl.ds(off, CHUNK)], buf2.at[b],
                             lsem[b])
            pltpu.make_async_copy(send_hbm.at[pl.ds(off, CHUNK)], idx2.at[b],
                                  lsem[b]).wait()
            pltpu.make_async_copy(ezh_hbm.at[:, pl.ds(off, CHUNK)],
                                  buf2.at[b], lsem[b]).wait()
            _repack(b)
            pltpu.async_copy(pay2.at[b], acc_sh.at[idx2.at[b]], ssem[b],
                             add=True)
        return carry

    lax.fori_loop(0, KFULL // 2, _pair, 0)
    for b in range(2):
        pltpu.make_async_copy(pay2.at[b], acc_sh.at[idx2.at[b]],
                              ssem[b]).wait()

    @pl.when(wid < NREM)
    def _():
        off = (KFULL * NW + wid) * CHUNK
        pltpu.sync_copy(send_hbm.at[pl.ds(off, CHUNK)], idx2.at[0])
        pltpu.sync_copy(ezh_hbm.at[:, pl.ds(off, CHUNK)], buf2.at[0])
        _repack(0)
        pltpu.sync_copy(pay2.at[0], acc_sh.at[idx2.at[0]], add=True)

    plsc.subcore_barrier()

    @pl.when(s < 15)
    def _():
        pltpu.sync_copy(acc_sh.at[pl.ds(s * RPT, RPT)],
                        part_hbm.at[c, pl.ds(s * RPT, RPT), :])

    @pl.when(s == 15)
    def _():
        pltpu.sync_copy(acc_sh.at[pl.ds(15 * RPT, RPT_LAST)],
                        part_hbm.at[c, pl.ds(15 * RPT, RPT_LAST), :])


def _k2b_body(p, o):
    o[...] = p[0] + p[1]


EPT = E // NW
CH3 = 2000
NCH3 = EPT // CH3
NG3 = CH3 // 16


@functools.partial(
    pl.kernel,
    out_type=jax.ShapeDtypeStruct((ROW, E), jnp.float32),
    mesh=_sc_mesh,
    compiler_params=_sc_params,
    scratch_types=[
        pltpu.VMEM((N * ROW,), jnp.float32),
        pltpu.VMEM((CH3,), jnp.int32),
        pltpu.VMEM((ROW, CH3), jnp.float32),
    ],
)
def _k3(send_hbm, denf_hbm, dgh_hbm, den_v, sidx_v, out_v):
    c = lax.axis_index("c")
    s = lax.axis_index("s")
    wid = s * 2 + c
    pltpu.sync_copy(denf_hbm, den_v)
    for ch in range(NCH3):
        ebase = wid * EPT + ch * CH3
        pltpu.sync_copy(send_hbm.at[pl.ds(ebase, CH3)], sidx_v)

        def _grp(g, carry):
            sv = sidx_v[pl.ds(g * 16, 16)] * ROW
            for j in range(ROW):
                gj = plsc.load_gather(den_v, (sv + j,))
                out_v[j, pl.ds(g * 16, 16)] = gj
            return carry

        lax.fori_loop(0, NG3, _grp, 0)
        pltpu.sync_copy(out_v, dgh_hbm.at[:, pl.ds(ebase, CH3)])


def _k4_body(xj, wt, ezh, dgh, hexp, hsum, m_out, mcm_out):
    t = jnp.dot(xj[...], wt[...], preferred_element_type=jnp.float32)
    row = lax.broadcasted_iota(jnp.int32, ezh.shape, 0)
    a = jnp.where(row < H, ezh[...] / dgh[...], 0.0)
    aexp = lax.dot_general(a, hexp[...], (((0,), (0,)), ((), ())),
                           preferred_element_type=jnp.float32)
    v = t * aexp
    m_out[...] = jnp.dot(v, hsum[...], preferred_element_type=jnp.float32)
    mcm_out[...] = lax.dot_general(hsum[...], v, (((0,), (1,)), ((), ())),
                                   preferred_element_type=jnp.float32)


NCH_SC = 16
KFULL5 = NCHUNK // 16
NREM5 = NCHUNK - KFULL5 * 16


@functools.partial(
    pl.kernel,
    out_type=jax.ShapeDtypeStruct((N, C), jnp.float32),
    mesh=_sc_mesh,
    compiler_params=_sc_params,
    scratch_types=[
        pltpu.VMEM((2, CHUNK), jnp.int32),
        pltpu.VMEM((2, NCH_SC, CHUNK), jnp.float32),
        pltpu.VMEM((2, CHUNK, NCH_SC), jnp.float32),
        pltpu.VMEM_SHARED((N, NCH_SC), jnp.float32),
        pltpu.SemaphoreType.DMA,
        pltpu.SemaphoreType.DMA,
        pltpu.SemaphoreType.DMA,
        pltpu.SemaphoreType.DMA,
    ],
)
def _k5(recv_hbm, mcm_hbm, zer16_hbm, aggr_hbm, idx2, buf2, pay2, acc_sh,
        lsem0, lsem1, ssem0, ssem1):
    c = lax.axis_index("c")
    s = lax.axis_index("s")
    colbase = c * NCH_SC
    lane = lax.iota(jnp.int32, 16)
    lsem = (lsem0, lsem1)
    ssem = (ssem0, ssem1)

    @pl.when(s < 15)
    def _():
        pltpu.sync_copy(zer16_hbm.at[pl.ds(s * RPT, RPT)],
                        acc_sh.at[pl.ds(s * RPT, RPT)])

    @pl.when(s == 15)
    def _():
        pltpu.sync_copy(zer16_hbm.at[pl.ds(15 * RPT, RPT_LAST)],
                        acc_sh.at[pl.ds(15 * RPT, RPT_LAST)])

    plsc.subcore_barrier()

    def _repack(b):
        def _rp(g, carry):
            rvec = g * 16 + lane
            for j in range(NCH_SC):
                v = buf2[b, j, pl.ds(g * 16, 16)]
                plsc.store_scatter(pay2.at[b],
                                   (rvec, jnp.full((16,), j, jnp.int32)), v)
            return carry

        lax.fori_loop(0, NG, _rp, 0)

    def _pair(g, carry):
        for b in range(2):
            k = 2 * g + b
            off = (k * 16 + s) * CHUNK

            @pl.when(g > 0)
            def _():
                pltpu.make_async_copy(pay2.at[b], acc_sh.at[idx2.at[b]],
                                      ssem[b]).wait()

            pltpu.async_copy(recv_hbm.at[pl.ds(off, CHUNK)], idx2.at[b],
                             lsem[b])
            pltpu.async_copy(
                mcm_hbm.at[pl.ds(colbase, NCH_SC), pl.ds(off, CHUNK)],
                buf2.at[b], lsem[b])
            pltpu.make_async_copy(recv_hbm.at[pl.ds(off, CHUNK)], idx2.at[b],
                                  lsem[b]).wait()
            pltpu.make_async_copy(
                mcm_hbm.at[pl.ds(colbase, NCH_SC), pl.ds(off, CHUNK)],
                buf2.at[b], lsem[b]).wait()
            _repack(b)
            pltpu.async_copy(pay2.at[b], acc_sh.at[idx2.at[b]], ssem[b],
                             add=True)
        return carry

    lax.fori_loop(0, KFULL5 // 2, _pair, 0)
    for b in range(2):
        pltpu.make_async_copy(pay2.at[b], acc_sh.at[idx2.at[b]],
                              ssem[b]).wait()

    @pl.when(s < NREM5)
    def _():
        off = ((KFULL5 * 16) + s) * CHUNK
        pltpu.sync_copy(recv_hbm.at[pl.ds(off, CHUNK)], idx2.at[0])
        pltpu.sync_copy(mcm_hbm.at[pl.ds(colbase, NCH_SC), pl.ds(off, CHUNK)],
                        buf2.at[0])
        _repack(0)
        pltpu.sync_copy(pay2.at[0], acc_sh.at[idx2.at[0]], add=True)

    plsc.subcore_barrier()

    @pl.when(s < 15)
    def _():
        pltpu.sync_copy(acc_sh.at[pl.ds(s * RPT, RPT)],
                        aggr_hbm.at[pl.ds(s * RPT, RPT), pl.ds(colbase, NCH_SC)])

    @pl.when(s == 15)
    def _():
        pltpu.sync_copy(acc_sh.at[pl.ds(15 * RPT, RPT_LAST)],
                        aggr_hbm.at[pl.ds(15 * RPT, RPT_LAST),
                                    pl.ds(colbase, NCH_SC)])


def kernel(x_i, x_j, edge_attribute, senders, receivers, Ws, Wt, We, attn):
    f32 = jnp.float32
    attn_flat = attn.reshape(1, HC)
    colid = jnp.arange(HC, dtype=jnp.int32)
    selt = (colid[None, :] // C ==
            jnp.arange(ROW, dtype=jnp.int32)[:, None]).astype(f32)
    hexp = selt
    hsum = ((colid % C)[:, None] ==
            jnp.arange(C, dtype=jnp.int32)[None, :]).astype(f32) * (1.0 / H)

    wspec = pl.BlockSpec((D, HC), lambda i: (0, 0))
    espec = pl.BlockSpec((ROW, BE), lambda i: (0, i))
    ezh = pl.pallas_call(
        _k1_body,
        grid=(GRID_E,),
        in_specs=[
            pl.BlockSpec((BE, D), lambda i: (i, 0)),
            pl.BlockSpec((BE, D), lambda i: (i, 0)),
            pl.BlockSpec((BE, D), lambda i: (i, 0)),
            wspec, wspec, wspec,
            pl.BlockSpec((1, HC), lambda i: (0, 0)),
            pl.BlockSpec((ROW, HC), lambda i: (0, 0)),
        ],
        out_specs=espec,
        out_shape=jax.ShapeDtypeStruct((ROW, E), f32),
    )(x_i, x_j, edge_attribute, Ws, Wt, We, attn_flat, selt)

    zer8 = jnp.zeros((N, ROW), f32)
    zer16 = jnp.zeros((N, NCH_SC), f32)

    partials = _k2(senders, ezh, zer8)

    denf = pl.pallas_call(
        _k2b_body,
        out_shape=jax.ShapeDtypeStruct((N * ROW // D, D), f32),
    )(partials.reshape(2, N * ROW // D, D)).reshape(N * ROW)

    dgh = _k3(senders, denf)

    m, mcm = pl.pallas_call(
        _k4_body,
        grid=(GRID_E,),
        in_specs=[
            pl.BlockSpec((BE, D), lambda i: (i, 0)),
            wspec,
            espec,
            espec,
            pl.BlockSpec((ROW, HC), lambda i: (0, 0)),
            pl.BlockSpec((HC, C), lambda i: (0, 0)),
        ],
        out_specs=[
            pl.BlockSpec((BE, C), lambda i: (i, 0)),
            pl.BlockSpec((C, BE), lambda i: (0, i)),
        ],
        out_shape=[
            jax.ShapeDtypeStruct((E, C), f32),
            jax.ShapeDtypeStruct((C, E), f32),
        ],
    )(x_j, Wt, ezh, dgh, hexp, hsum)

    aggr = _k5(receivers, mcm, zer16)

    return (aggr, m)


# speedup vs baseline: 1.6495x; 1.0291x over previous
import functools

import jax
import jax.numpy as jnp
from jax import lax
from jax.experimental import pallas as pl
from jax.experimental.pallas import tpu as pltpu
from jax.experimental.pallas import tpu_sc as plsc

E = 320000
N = 10000
H = 4
C = 32
D = 128
HC = H * C

ROW = 8
CHUNK = 128
NG = CHUNK // 16
NCHUNK = E // CHUNK
NW = 32
KFULL = NCHUNK // NW
NREM = NCHUNK - KFULL * NW
RPT = 632
RPT_LAST = N - 15 * RPT

BE = 2560
GRID_E = E // BE

_sc_mesh = plsc.VectorSubcoreMesh(core_axis_name="c", subcore_axis_name="s",
                                  num_cores=2, num_subcores=16)
_sc_params = pltpu.CompilerParams(use_tc_tiling_on_sc=False,
                                  needs_layout_passes=False)


def _k1_body(xi, xj, ea, ws, wt, we, attn, selt, ez_out):
    u = jnp.dot(xj[...], wt[...], preferred_element_type=jnp.float32)
    u = u + jnp.dot(xi[...], ws[...], preferred_element_type=jnp.float32)
    u = u + jnp.dot(ea[...], we[...], preferred_element_type=jnp.float32)
    u = jnp.where(u >= 0.0, u, 0.01 * u) * attn[...]
    zt = lax.dot_general(selt[...], u, (((1,), (1,)), ((), ())),
                         preferred_element_type=jnp.float32)
    row = lax.broadcasted_iota(jnp.int32, zt.shape, 0)
    ez_out[...] = jnp.where(row < H, jnp.exp(zt), 0.0)


@functools.partial(
    pl.kernel,
    out_type=jax.ShapeDtypeStruct((2, N, ROW), jnp.float32),
    mesh=_sc_mesh,
    compiler_params=_sc_params,
    scratch_types=[
        pltpu.VMEM((2, CHUNK), jnp.int32),
        pltpu.VMEM((2, ROW, CHUNK), jnp.float32),
        pltpu.VMEM((2, CHUNK, ROW), jnp.float32),
        pltpu.VMEM_SHARED((N, ROW), jnp.float32),
        pltpu.SemaphoreType.DMA,
        pltpu.SemaphoreType.DMA,
        pltpu.SemaphoreType.DMA,
        pltpu.SemaphoreType.DMA,
    ],
)
def _k2(send_hbm, ezh_hbm, zer8_hbm, part_hbm, idx2, buf2, pay2, acc_sh,
        lsem0, lsem1, ssem0, ssem1):
    c = lax.axis_index("c")
    s = lax.axis_index("s")
    wid = s * 2 + c
    lane = lax.iota(jnp.int32, 16)
    lsem = (lsem0, lsem1)
    ssem = (ssem0, ssem1)

    @pl.when(s < 15)
    def _():
        pltpu.sync_copy(zer8_hbm.at[pl.ds(s * RPT, RPT)],
                        acc_sh.at[pl.ds(s * RPT, RPT)])

    @pl.when(s == 15)
    def _():
        pltpu.sync_copy(zer8_hbm.at[pl.ds(15 * RPT, RPT_LAST)],
                        acc_sh.at[pl.ds(15 * RPT, RPT_LAST)])

    plsc.subcore_barrier()

    def _repack(b):
        def _rp(g, carry):
            rvec = g * 16 + lane
            for j in range(ROW):
                v = buf2[b, j, pl.ds(g * 16, 16)]
                plsc.store_scatter(pay2.at[b],
                                   (rvec, jnp.full((16,), j, jnp.int32)), v)
            return carry

        lax.fori_loop(0, NG, _rp, 0)

    def _pair(g, carry):
        for b in range(2):
            k = 2 * g + b
            off = (k * NW + wid) * CHUNK

            @pl.when(g > 0)
            def _():
                pltpu.make_async_copy(pay2.at[b], acc_sh.at[idx2.at[b]],
                                      ssem[b]).wait()

            pltpu.async_copy(send_hbm.at[pl.ds(off, CHUNK)], idx2.at[b],
                             lsem[b])
            pltpu.async_copy(ezh_hbm.at[:, pl.ds(off, CHUNK)], buf2.at[b],
                             lsem[b])
            pltpu.make_async_copy(send_hbm.at[pl.ds(off, CHUNK)], idx2.at[b],
                                  lsem[b]).wait()
            pltpu.make_async_copy(ezh_hbm.at[:, pl.ds(off, CHUNK)],
                                  buf2.at[b], lsem[b]).wait()
            _repack(b)
            pltpu.async_copy(pay2.at[b], acc_sh.at[idx2.at[b]], ssem[b],
                             add=True)
        return carry

    lax.fori_loop(0, KFULL // 2, _pair, 0)
    for b in range(2):
        pltpu.make_async_copy(pay2.at[b], acc_sh.at[idx2.at[b]],
                              ssem[b]).wait()

    @pl.when(wid < NREM)
    def _():
        off = (KFULL * NW + wid) * CHUNK
        pltpu.sync_copy(send_hbm.at[pl.ds(off, CHUNK)], idx2.at[0])
        pltpu.sync_copy(ezh_hbm.at[:, pl.ds(off, CHUNK)], buf2.at[0])
        _repack(0)
        pltpu.sync_copy(pay2.at[0], acc_sh.at[idx2.at[0]], add=True)

    plsc.subcore_barrier()

    @pl.when(s < 15)
    def _():
        pltpu.sync_copy(acc_sh.at[pl.ds(s * RPT, RPT)],
                        part_hbm.at[c, pl.ds(s * RPT, RPT), :])

    @pl.when(s == 15)
    def _():
        pltpu.sync_copy(acc_sh.at[pl.ds(15 * RPT, RPT_LAST)],
                        part_hbm.at[c, pl.ds(15 * RPT, RPT_LAST), :])


def _k2b_body(p, o):
    o[...] = p[0] + p[1]


EPT = E // NW
CH3 = 2000
NCH3 = EPT // CH3
NG3 = CH3 // 16


@functools.partial(
    pl.kernel,
    out_type=jax.ShapeDtypeStruct((ROW, E), jnp.float32),
    mesh=_sc_mesh,
    compiler_params=_sc_params,
    scratch_types=[
        pltpu.VMEM((N * ROW,), jnp.float32),
        pltpu.VMEM((CH3,), jnp.int32),
        pltpu.VMEM((ROW, CH3), jnp.float32),
    ],
)
def _k3(send_hbm, denf_hbm, dgh_hbm, den_v, sidx_v, out_v):
    c = lax.axis_index("c")
    s = lax.axis_index("s")
    wid = s * 2 + c
    pltpu.sync_copy(denf_hbm, den_v)
    for ch in range(NCH3):
        ebase = wid * EPT + ch * CH3
        pltpu.sync_copy(send_hbm.at[pl.ds(ebase, CH3)], sidx_v)

        def _grp(g, carry):
            sv = sidx_v[pl.ds(g * 16, 16)] * ROW
            for j in range(ROW):
                gj = plsc.load_gather(den_v, (sv + j,))
                out_v[j, pl.ds(g * 16, 16)] = gj
            return carry

        lax.fori_loop(0, NG3, _grp, 0)
        pltpu.sync_copy(out_v, dgh_hbm.at[:, pl.ds(ebase, CH3)])


def _k4_body(xj, wt, ezh, dgh, hexp, hsum, m_out, mcm_out):
    t = jnp.dot(xj[...], wt[...], preferred_element_type=jnp.float32)
    row = lax.broadcasted_iota(jnp.int32, ezh.shape, 0)
    a = jnp.where(row < H, ezh[...] / dgh[...], 0.0)
    aexp = lax.dot_general(a, hexp[...], (((0,), (0,)), ((), ())),
                           preferred_element_type=jnp.float32)
    v = t * aexp
    m_out[...] = jnp.dot(v, hsum[...], preferred_element_type=jnp.float32)
    for g in range(4):
        mg = lax.dot_general(hsum[...][:, g * 8:(g + 1) * 8], v,
                             (((0,), (1,)), ((), ())),
                             preferred_element_type=jnp.float32)
        mcm_out[g] = jnp.transpose(mg.reshape(8, BE // D, D), (1, 0, 2))


NCH_SC = 16
KFULL5 = NCHUNK // 16
NREM5 = NCHUNK - KFULL5 * 16


@functools.partial(
    pl.kernel,
    out_type=jax.ShapeDtypeStruct((N, C), jnp.float32),
    mesh=_sc_mesh,
    compiler_params=_sc_params,
    scratch_types=[
        pltpu.VMEM((2, CHUNK), jnp.int32),
        pltpu.VMEM((2, 2, ROW, CHUNK), jnp.float32),
        pltpu.VMEM((2, CHUNK, NCH_SC), jnp.float32),
        pltpu.VMEM_SHARED((N, NCH_SC), jnp.float32),
        pltpu.SemaphoreType.DMA,
        pltpu.SemaphoreType.DMA,
        pltpu.SemaphoreType.DMA,
        pltpu.SemaphoreType.DMA,
    ],
)
def _k5(recv_hbm, mcm_hbm, zer16_hbm, aggr_hbm, idx2, buf2, pay2, acc_sh,
        lsem0, lsem1, ssem0, ssem1):
    c = lax.axis_index("c")
    s = lax.axis_index("s")
    colbase = c * NCH_SC
    lane = lax.iota(jnp.int32, 16)
    lsem = (lsem0, lsem1)
    ssem = (ssem0, ssem1)

    @pl.when(s < 15)
    def _():
        pltpu.sync_copy(zer16_hbm.at[pl.ds(s * RPT, RPT)],
                        acc_sh.at[pl.ds(s * RPT, RPT)])

    @pl.when(s == 15)
    def _():
        pltpu.sync_copy(zer16_hbm.at[pl.ds(15 * RPT, RPT_LAST)],
                        acc_sh.at[pl.ds(15 * RPT, RPT_LAST)])

    plsc.subcore_barrier()

    def _repack(b):
        def _rp(g, carry):
            rvec = g * 16 + lane
            for gg in range(2):
                for r in range(ROW):
                    v = buf2[b, gg, r, pl.ds(g * 16, 16)]
                    plsc.store_scatter(
                        pay2.at[b],
                        (rvec, jnp.full((16,), gg * ROW + r, jnp.int32)), v)
            return carry

        lax.fori_loop(0, NG, _rp, 0)

    def _one_load(b, k, sync):
        chunk = k * 16 + s
        off = chunk * CHUNK
        if sync:
            pltpu.sync_copy(recv_hbm.at[pl.ds(off, CHUNK)], idx2.at[b])
            pltpu.sync_copy(mcm_hbm.at[pl.ds(2 * c, 2), chunk, :, :],
                            buf2.at[b])
        else:
            pltpu.async_copy(recv_hbm.at[pl.ds(off, CHUNK)], idx2.at[b],
                             lsem[b])
            pltpu.async_copy(mcm_hbm.at[pl.ds(2 * c, 2), chunk, :, :],
                             buf2.at[b], lsem[b])
            pltpu.make_async_copy(recv_hbm.at[pl.ds(off, CHUNK)], idx2.at[b],
                                  lsem[b]).wait()
            pltpu.make_async_copy(mcm_hbm.at[pl.ds(2 * c, 2), chunk, :, :],
                                  buf2.at[b], lsem[b]).wait()

    def _pair(g, carry):
        for b in range(2):
            k = 2 * g + b

            @pl.when(g > 0)
            def _():
                pltpu.make_async_copy(pay2.at[b], acc_sh.at[idx2.at[b]],
                                      ssem[b]).wait()

            _one_load(b, k, False)
            _repack(b)
            pltpu.async_copy(pay2.at[b], acc_sh.at[idx2.at[b]], ssem[b],
                             add=True)
        return carry

    lax.fori_loop(0, KFULL5 // 2, _pair, 0)
    for b in range(2):
        pltpu.make_async_copy(pay2.at[b], acc_sh.at[idx2.at[b]],
                              ssem[b]).wait()

    @pl.when(s < NREM5)
    def _():
        _one_load(0, KFULL5, True)
        _repack(0)
        pltpu.sync_copy(pay2.at[0], acc_sh.at[idx2.at[0]], add=True)

    plsc.subcore_barrier()

    @pl.when(s < 15)
    def _():
        pltpu.sync_copy(acc_sh.at[pl.ds(s * RPT, RPT)],
                        aggr_hbm.at[pl.ds(s * RPT, RPT), pl.ds(colbase, NCH_SC)])

    @pl.when(s == 15)
    def _():
        pltpu.sync_copy(acc_sh.at[pl.ds(15 * RPT, RPT_LAST)],
                        aggr_hbm.at[pl.ds(15 * RPT, RPT_LAST),
                                    pl.ds(colbase, NCH_SC)])


def kernel(x_i, x_j, edge_attribute, senders, receivers, Ws, Wt, We, attn):
    f32 = jnp.float32
    attn_flat = attn.reshape(1, HC)
    colid = jnp.arange(HC, dtype=jnp.int32)
    selt = (colid[None, :] // C ==
            jnp.arange(ROW, dtype=jnp.int32)[:, None]).astype(f32)
    hexp = selt
    hsum = ((colid % C)[:, None] ==
            jnp.arange(C, dtype=jnp.int32)[None, :]).astype(f32) * (1.0 / H)

    wspec = pl.BlockSpec((D, HC), lambda i: (0, 0))
    espec = pl.BlockSpec((ROW, BE), lambda i: (0, i))
    ezh = pl.pallas_call(
        _k1_body,
        grid=(GRID_E,),
        in_specs=[
            pl.BlockSpec((BE, D), lambda i: (i, 0)),
            pl.BlockSpec((BE, D), lambda i: (i, 0)),
            pl.BlockSpec((BE, D), lambda i: (i, 0)),
            wspec, wspec, wspec,
            pl.BlockSpec((1, HC), lambda i: (0, 0)),
            pl.BlockSpec((ROW, HC), lambda i: (0, 0)),
        ],
        out_specs=espec,
        out_shape=jax.ShapeDtypeStruct((ROW, E), f32),
    )(x_i, x_j, edge_attribute, Ws, Wt, We, attn_flat, selt)

    zer8 = jnp.zeros((N, ROW), f32)
    zer16 = jnp.zeros((N, NCH_SC), f32)

    partials = _k2(senders, ezh, zer8)

    denf = pl.pallas_call(
        _k2b_body,
        out_shape=jax.ShapeDtypeStruct((N * ROW // D, D), f32),
    )(partials.reshape(2, N * ROW // D, D)).reshape(N * ROW)

    dgh = _k3(senders, denf)

    m, mcm = pl.pallas_call(
        _k4_body,
        grid=(GRID_E,),
        in_specs=[
            pl.BlockSpec((BE, D), lambda i: (i, 0)),
            wspec,
            espec,
            espec,
            pl.BlockSpec((ROW, HC), lambda i: (0, 0)),
            pl.BlockSpec((HC, C), lambda i: (0, 0)),
        ],
        out_specs=[
            pl.BlockSpec((BE, C), lambda i: (i, 0)),
            pl.BlockSpec((4, BE // D, ROW, D), lambda i: (0, i, 0, 0)),
        ],
        out_shape=[
            jax.ShapeDtypeStruct((E, C), f32),
            jax.ShapeDtypeStruct((4, NCHUNK, ROW, D), f32),
        ],
    )(x_j, Wt, ezh, dgh, hexp, hsum)

    aggr = _k5(receivers, mcm, zer16)

    return (aggr, m)


# speedup vs baseline: 1.7416x; 1.0558x over previous
import functools

import jax
import jax.numpy as jnp
from jax import lax
from jax.experimental import pallas as pl
from jax.experimental.pallas import tpu as pltpu
from jax.experimental.pallas import tpu_sc as plsc

E = 320000
N = 10000
H = 4
C = 32
D = 128
HC = H * C

ROW = 8
CHUNK = 128
NG = CHUNK // 16
NCHUNK = E // CHUNK
NW = 32
KFULL = NCHUNK // NW
NREM = NCHUNK - KFULL * NW
RPT = 632
RPT_LAST = N - 15 * RPT

BE = 2560
GRID_E = E // BE

_sc_mesh = plsc.VectorSubcoreMesh(core_axis_name="c", subcore_axis_name="s",
                                  num_cores=2, num_subcores=16)
_sc_params = pltpu.CompilerParams(use_tc_tiling_on_sc=False,
                                  needs_layout_passes=False)


def _k1_body(xi, xj, ea, ws, wt, we, attn, selt, ez_out):
    u = jnp.dot(xj[...], wt[...], preferred_element_type=jnp.float32)
    u = u + jnp.dot(xi[...], ws[...], preferred_element_type=jnp.float32)
    u = u + jnp.dot(ea[...], we[...], preferred_element_type=jnp.float32)
    u = jnp.where(u >= 0.0, u, 0.01 * u) * attn[...]
    zt = lax.dot_general(selt[...], u, (((1,), (1,)), ((), ())),
                         preferred_element_type=jnp.float32)
    row = lax.broadcasted_iota(jnp.int32, zt.shape, 0)
    ez_out[...] = jnp.where(row < H, jnp.exp(zt), 0.0)


@functools.partial(
    pl.kernel,
    out_type=jax.ShapeDtypeStruct((2, N, ROW), jnp.float32),
    mesh=_sc_mesh,
    compiler_params=_sc_params,
    scratch_types=[
        pltpu.VMEM((2, CHUNK), jnp.int32),
        pltpu.VMEM((2, ROW, CHUNK), jnp.float32),
        pltpu.VMEM((2, CHUNK, ROW), jnp.float32),
        pltpu.VMEM_SHARED((N, ROW), jnp.float32),
        pltpu.SemaphoreType.DMA,
        pltpu.SemaphoreType.DMA,
        pltpu.SemaphoreType.DMA,
        pltpu.SemaphoreType.DMA,
    ],
)
def _k2(send_hbm, ezh_hbm, zer8_hbm, part_hbm, idx2, buf2, pay2, acc_sh,
        lsem0, lsem1, ssem0, ssem1):
    c = lax.axis_index("c")
    s = lax.axis_index("s")
    wid = s * 2 + c
    lane = lax.iota(jnp.int32, 16)
    lsem = (lsem0, lsem1)
    ssem = (ssem0, ssem1)

    @pl.when(s < 15)
    def _():
        pltpu.sync_copy(zer8_hbm.at[pl.ds(s * RPT, RPT)],
                        acc_sh.at[pl.ds(s * RPT, RPT)])

    @pl.when(s == 15)
    def _():
        pltpu.sync_copy(zer8_hbm.at[pl.ds(15 * RPT, RPT_LAST)],
                        acc_sh.at[pl.ds(15 * RPT, RPT_LAST)])

    plsc.subcore_barrier()

    def _repack(b):
        def _rp(g, carry):
            rvec = g * 16 + lane
            for j in range(ROW):
                v = buf2[b, j, pl.ds(g * 16, 16)]
                plsc.store_scatter(pay2.at[b],
                                   (rvec, jnp.full((16,), j, jnp.int32)), v)
            return carry

        lax.fori_loop(0, NG, _rp, 0)

    def _pair(g, carry):
        for b in range(2):
            k = 2 * g + b
            off = (k * NW + wid) * CHUNK

            @pl.when(g > 0)
            def _():
                pltpu.make_async_copy(pay2.at[b], acc_sh.at[idx2.at[b]],
                                      ssem[b]).wait()

            pltpu.async_copy(send_hbm.at[pl.ds(off, CHUNK)], idx2.at[b],
                             lsem[b])
            pltpu.async_copy(ezh_hbm.at[:, pl.ds(off, CHUNK)], buf2.at[b],
                             lsem[b])
            pltpu.make_async_copy(send_hbm.at[pl.ds(off, CHUNK)], idx2.at[b],
                                  lsem[b]).wait()
            pltpu.make_async_copy(ezh_hbm.at[:, pl.ds(off, CHUNK)],
                                  buf2.at[b], lsem[b]).wait()
            _repack(b)
            pltpu.async_copy(pay2.at[b], acc_sh.at[idx2.at[b]], ssem[b],
                             add=True)
        return carry

    lax.fori_loop(0, KFULL // 2, _pair, 0)
    for b in range(2):
        pltpu.make_async_copy(pay2.at[b], acc_sh.at[idx2.at[b]],
                              ssem[b]).wait()

    @pl.when(wid < NREM)
    def _():
        off = (KFULL * NW + wid) * CHUNK
        pltpu.sync_copy(send_hbm.at[pl.ds(off, CHUNK)], idx2.at[0])
        pltpu.sync_copy(ezh_hbm.at[:, pl.ds(off, CHUNK)], buf2.at[0])
        _repack(0)
        pltpu.sync_copy(pay2.at[0], acc_sh.at[idx2.at[0]], add=True)

    plsc.subcore_barrier()

    @pl.when(s < 15)
    def _():
        pltpu.sync_copy(acc_sh.at[pl.ds(s * RPT, RPT)],
                        part_hbm.at[c, pl.ds(s * RPT, RPT), :])

    @pl.when(s == 15)
    def _():
        pltpu.sync_copy(acc_sh.at[pl.ds(15 * RPT, RPT_LAST)],
                        part_hbm.at[c, pl.ds(15 * RPT, RPT_LAST), :])


def _k2b_body(p, o):
    o[...] = p[0] + p[1]


EPT = E // NW
CH3 = 2000
NCH3 = EPT // CH3
NG3 = CH3 // 16


@functools.partial(
    pl.kernel,
    out_type=jax.ShapeDtypeStruct((ROW, E), jnp.float32),
    mesh=_sc_mesh,
    compiler_params=_sc_params,
    scratch_types=[
        pltpu.VMEM((N * ROW,), jnp.float32),
        pltpu.VMEM((CH3,), jnp.int32),
        pltpu.VMEM((ROW, CH3), jnp.float32),
    ],
)
def _k3(send_hbm, denf_hbm, dgh_hbm, den_v, sidx_v, out_v):
    c = lax.axis_index("c")
    s = lax.axis_index("s")
    wid = s * 2 + c
    pltpu.sync_copy(denf_hbm, den_v)
    for ch in range(NCH3):
        ebase = wid * EPT + ch * CH3
        pltpu.sync_copy(send_hbm.at[pl.ds(ebase, CH3)], sidx_v)

        def _grp(g, carry):
            sv = sidx_v[pl.ds(g * 16, 16)] * ROW
            for j in range(ROW):
                gj = plsc.load_gather(den_v, (sv + j,))
                out_v[j, pl.ds(g * 16, 16)] = gj
            return carry

        lax.fori_loop(0, NG3, _grp, 0)
        pltpu.sync_copy(out_v, dgh_hbm.at[:, pl.ds(ebase, CH3)])


def _k4_body(xj, wt, ezh, dgh, hexp, hsum, mcm_out):
    t = jnp.dot(xj[...], wt[...], preferred_element_type=jnp.float32)
    row = lax.broadcasted_iota(jnp.int32, ezh.shape, 0)
    a = jnp.where(row < H, ezh[...] / dgh[...], 0.0)
    aexp = lax.dot_general(a, hexp[...], (((0,), (0,)), ((), ())),
                           preferred_element_type=jnp.float32)
    v = t * aexp
    for g in range(4):
        mg = lax.dot_general(hsum[...][:, g * 8:(g + 1) * 8], v,
                             (((0,), (1,)), ((), ())),
                             preferred_element_type=jnp.float32)
        mcm_out[g] = jnp.transpose(mg.reshape(8, BE // D, D), (1, 0, 2))


NCH_SC = 16
KFULL5 = NCHUNK // 16
NREM5 = NCHUNK - KFULL5 * 16


@functools.partial(
    pl.kernel,
    out_type=jax.ShapeDtypeStruct((N, C), jnp.float32),
    mesh=_sc_mesh,
    compiler_params=_sc_params,
    scratch_types=[
        pltpu.VMEM((2, CHUNK), jnp.int32),
        pltpu.VMEM((2, 2, ROW, CHUNK), jnp.float32),
        pltpu.VMEM((2, CHUNK, NCH_SC), jnp.float32),
        pltpu.VMEM_SHARED((N, NCH_SC), jnp.float32),
        pltpu.SemaphoreType.DMA,
        pltpu.SemaphoreType.DMA,
        pltpu.SemaphoreType.DMA,
        pltpu.SemaphoreType.DMA,
    ],
)
def _k5(recv_hbm, mcm_hbm, zer16_hbm, aggr_hbm, idx2, buf2, pay2, acc_sh,
        lsem0, lsem1, ssem0, ssem1):
    c = lax.axis_index("c")
    s = lax.axis_index("s")
    colbase = c * NCH_SC
    lane = lax.iota(jnp.int32, 16)
    lsem = (lsem0, lsem1)
    ssem = (ssem0, ssem1)

    @pl.when(s < 15)
    def _():
        pltpu.sync_copy(zer16_hbm.at[pl.ds(s * RPT, RPT)],
                        acc_sh.at[pl.ds(s * RPT, RPT)])

    @pl.when(s == 15)
    def _():
        pltpu.sync_copy(zer16_hbm.at[pl.ds(15 * RPT, RPT_LAST)],
                        acc_sh.at[pl.ds(15 * RPT, RPT_LAST)])

    plsc.subcore_barrier()

    def _repack(b):
        def _rp(g, carry):
            rvec = g * 16 + lane
            for gg in range(2):
                for r in range(ROW):
                    v = buf2[b, gg, r, pl.ds(g * 16, 16)]
                    plsc.store_scatter(
                        pay2.at[b],
                        (rvec, jnp.full((16,), gg * ROW + r, jnp.int32)), v)
            return carry

        lax.fori_loop(0, NG, _rp, 0)

    def _one_load(b, k, sync):
        chunk = k * 16 + s
        off = chunk * CHUNK
        if sync:
            pltpu.sync_copy(recv_hbm.at[pl.ds(off, CHUNK)], idx2.at[b])
            pltpu.sync_copy(mcm_hbm.at[pl.ds(2 * c, 2), chunk, :, :],
                            buf2.at[b])
        else:
            pltpu.async_copy(recv_hbm.at[pl.ds(off, CHUNK)], idx2.at[b],
                             lsem[b])
            pltpu.async_copy(mcm_hbm.at[pl.ds(2 * c, 2), chunk, :, :],
                             buf2.at[b], lsem[b])
            pltpu.make_async_copy(recv_hbm.at[pl.ds(off, CHUNK)], idx2.at[b],
                                  lsem[b]).wait()
            pltpu.make_async_copy(mcm_hbm.at[pl.ds(2 * c, 2), chunk, :, :],
                                  buf2.at[b], lsem[b]).wait()

    def _pair(g, carry):
        for b in range(2):
            k = 2 * g + b

            @pl.when(g > 0)
            def _():
                pltpu.make_async_copy(pay2.at[b], acc_sh.at[idx2.at[b]],
                                      ssem[b]).wait()

            _one_load(b, k, False)
            _repack(b)
            pltpu.async_copy(pay2.at[b], acc_sh.at[idx2.at[b]], ssem[b],
                             add=True)
        return carry

    lax.fori_loop(0, KFULL5 // 2, _pair, 0)
    for b in range(2):
        pltpu.make_async_copy(pay2.at[b], acc_sh.at[idx2.at[b]],
                              ssem[b]).wait()

    @pl.when(s < NREM5)
    def _():
        _one_load(0, KFULL5, True)
        _repack(0)
        pltpu.sync_copy(pay2.at[0], acc_sh.at[idx2.at[0]], add=True)

    plsc.subcore_barrier()

    @pl.when(s < 15)
    def _():
        pltpu.sync_copy(acc_sh.at[pl.ds(s * RPT, RPT)],
                        aggr_hbm.at[pl.ds(s * RPT, RPT), pl.ds(colbase, NCH_SC)])

    @pl.when(s == 15)
    def _():
        pltpu.sync_copy(acc_sh.at[pl.ds(15 * RPT, RPT_LAST)],
                        aggr_hbm.at[pl.ds(15 * RPT, RPT_LAST),
                                    pl.ds(colbase, NCH_SC)])


def kernel(x_i, x_j, edge_attribute, senders, receivers, Ws, Wt, We, attn):
    f32 = jnp.float32
    attn_flat = attn.reshape(1, HC)
    colid = jnp.arange(HC, dtype=jnp.int32)
    selt = (colid[None, :] // C ==
            jnp.arange(ROW, dtype=jnp.int32)[:, None]).astype(f32)
    hexp = selt
    hsum = ((colid % C)[:, None] ==
            jnp.arange(C, dtype=jnp.int32)[None, :]).astype(f32) * (1.0 / H)

    wspec = pl.BlockSpec((D, HC), lambda i: (0, 0))
    espec = pl.BlockSpec((ROW, BE), lambda i: (0, i))
    ezh = pl.pallas_call(
        _k1_body,
        grid=(GRID_E,),
        in_specs=[
            pl.BlockSpec((BE, D), lambda i: (i, 0)),
            pl.BlockSpec((BE, D), lambda i: (i, 0)),
            pl.BlockSpec((BE, D), lambda i: (i, 0)),
            wspec, wspec, wspec,
            pl.BlockSpec((1, HC), lambda i: (0, 0)),
            pl.BlockSpec((ROW, HC), lambda i: (0, 0)),
        ],
        out_specs=espec,
        out_shape=jax.ShapeDtypeStruct((ROW, E), f32),
    )(x_i, x_j, edge_attribute, Ws, Wt, We, attn_flat, selt)

    zer8 = jnp.zeros((N, ROW), f32)
    zer16 = jnp.zeros((N, NCH_SC), f32)

    partials = _k2(senders, ezh, zer8)

    denf = pl.pallas_call(
        _k2b_body,
        out_shape=jax.ShapeDtypeStruct((N * ROW // D, D), f32),
    )(partials.reshape(2, N * ROW // D, D)).reshape(N * ROW)

    dgh = _k3(senders, denf)

    mcm = pl.pallas_call(
        _k4_body,
        grid=(GRID_E,),
        in_specs=[
            pl.BlockSpec((BE, D), lambda i: (i, 0)),
            wspec,
            espec,
            espec,
            pl.BlockSpec((ROW, HC), lambda i: (0, 0)),
            pl.BlockSpec((HC, C), lambda i: (0, 0)),
        ],
        out_specs=pl.BlockSpec((4, BE // D, ROW, D), lambda i: (0, i, 0, 0)),
        out_shape=jax.ShapeDtypeStruct((4, NCHUNK, ROW, D), f32),
    )(x_j, Wt, ezh, dgh, hexp, hsum)

    aggr = _k5(receivers, mcm, zer16)

    m = mcm.transpose(0, 2, 1, 3).reshape(C, E).T

    return (aggr, m)
